# baseline (speedup 1.0000x reference)
import functools

import jax
import jax.numpy as jnp
import numpy as np
from jax.experimental import pallas as pl
from jax.experimental.pallas import tpu as pltpu

B, S, H, L, NH, DH = 1, 2048, 768, 2, 12, 64
V, TV, FF, BS, NR = 30522, 2, 3072, 64, 3
NB = S // BS
NK = 5 + NR


def _block_layout(nb, num_rand, seed):
    rng = np.random.RandomState(seed)
    idx = np.zeros((nb, 5 + num_rand), np.int32)
    valid = np.ones((nb, 5 + num_rand), np.float32)
    for i in range(nb):
        fixed = [0, nb - 1, (i - 1) % nb, i, (i + 1) % nb]
        rem = sorted(set(range(nb)) - set(fixed))
        r = rng.choice(rem, num_rand, replace=False)
        row = fixed + list(r)
        seen = set()
        for j, c in enumerate(row):
            idx[i, j] = c
            if c in seen:
                valid[i, j] = 0.0
            seen.add(c)
    return idx, valid


_LAYOUTS = [_block_layout(NB, NR, i) for i in range(L)]

def _ln(x, g, b):
    m = jnp.mean(x, -1, keepdims=True)
    v = jnp.mean((x - m) * (x - m), -1, keepdims=True)
    return (x - m) / jnp.sqrt(v + 1e-12) * g + b


def _gembed_body(ids_ref, tab_ref, pos_ref, tid_ref, te_ref, gg_ref, bb_ref,
                 o_ref, gath_ref, sem):
    def issue(t, _):
        for u in range(8):
            pltpu.make_async_copy(
                tab_ref.at[pl.ds(ids_ref[t * 8 + u], 1), :],
                gath_ref.at[pl.ds(t * 8 + u, 1), :],
                sem,
            ).start()
        return 0

    jax.lax.fori_loop(0, S // 8, issue, 0)

    def wait(t, _):
        pltpu.make_async_copy(
            tab_ref.at[pl.ds(0, 1), :], gath_ref.at[pl.ds(0, 1), :], sem
        ).wait()
        return 0

    jax.lax.fori_loop(0, S, wait, 0)

    x = gath_ref[...] + pos_ref[...]
    cond = tid_ref[...] == 0
    x = x + jnp.where(cond, te_ref[0:1, :], te_ref[1:2, :])
    o_ref[...] = _ln(x, gg_ref[...], bb_ref[...])


def _gembed(word_ids, word_emb, pos_emb, type_ids, type_emb, g, b):
    grid_spec = pltpu.PrefetchScalarGridSpec(
        num_scalar_prefetch=1,
        grid=(1,),
        in_specs=[
            pl.BlockSpec(memory_space=pl.ANY),
            pl.BlockSpec((S, H), lambda i, *_: (0, 0)),
            pl.BlockSpec((S, 1), lambda i, *_: (0, 0)),
            pl.BlockSpec((TV, H), lambda i, *_: (0, 0)),
            pl.BlockSpec((1, H), lambda i, *_: (0, 0)),
            pl.BlockSpec((1, H), lambda i, *_: (0, 0)),
        ],
        out_specs=pl.BlockSpec((S, H), lambda i, *_: (0, 0)),
        scratch_shapes=[
            pltpu.VMEM((S, H), jnp.float32),
            pltpu.SemaphoreType.DMA,
        ],
    )
    return pl.pallas_call(
        _gembed_body,
        grid_spec=grid_spec,
        out_shape=jax.ShapeDtypeStruct((S, H), jnp.float32),
    )(word_ids, word_emb, pos_emb, type_ids, type_emb, g, b)


_QB = 256


def _qkv_body(x_ref, w_ref, b_ref, q_ref, kt_ref, v_ref):
    xb = x_ref[...].astype(jnp.bfloat16)
    r = jax.lax.dot_general(
        xb, w_ref[...], (((1,), (0,)), ((), ())),
        preferred_element_type=jnp.float32,
    )
    r = (r + b_ref[...]).astype(jnp.bfloat16)
    for h in range(NH):
        q_ref[h] = r[:, h * DH:(h + 1) * DH]
        v_ref[h] = r[:, 2 * H + h * DH:2 * H + (h + 1) * DH]
        for sb in range(_QB // BS):
            kt_ref[sb, h] = jnp.transpose(
                r[sb * BS:(sb + 1) * BS, H + h * DH:H + (h + 1) * DH])


def _qkv(x, wqkv, bqkv):
    grid = (S // _QB,)
    return pl.pallas_call(
        _qkv_body,
        grid=grid,
        in_specs=[
            pl.BlockSpec((_QB, H), lambda i: (i, 0)),
            pl.BlockSpec((H, 3 * H), lambda i: (0, 0)),
            pl.BlockSpec((1, 3 * H), lambda i: (0, 0)),
        ],
        out_specs=[
            pl.BlockSpec((NH, _QB, DH), lambda i: (0, i, 0)),
            pl.BlockSpec((_QB // BS, NH, DH, BS), lambda i: (i, 0, 0, 0)),
            pl.BlockSpec((NH, _QB, DH), lambda i: (0, i, 0)),
        ],
        out_shape=[
            jax.ShapeDtypeStruct((NH, S, DH), jnp.bfloat16),
            jax.ShapeDtypeStruct((NB, NH, DH, BS), jnp.bfloat16),
            jax.ShapeDtypeStruct((NH, S, DH), jnp.bfloat16),
        ],
    )(x, wqkv, bqkv)


def _attn_body(idx_ref, val_ref, q_ref, kt_ref, v_ref, mask_ref, o_ref,
               kgt_ref, vg_ref):
    n = pl.program_id(0)
    bias_parts = []
    for j in range(NK):
        bi = idx_ref[n * NK + j]
        kgt_ref[:, :, j * BS:(j + 1) * BS] = kt_ref[bi]
        vg_ref[:, j * BS:(j + 1) * BS, :] = v_ref[:, pl.ds(bi * BS, BS), :]
        mv = mask_ref[bi]
        vj = val_ref[n * NK + j].astype(jnp.float32)
        bias_parts.append((1.0 - mv * vj) * (-1e9))
    bias = jnp.concatenate(bias_parts, axis=-1)[None]

    qb = q_ref[...]
    s = jax.lax.dot_general(
        qb, kgt_ref[...], (((2,), (1,)), ((0,), (0,))),
        preferred_element_type=jnp.float32,
    )
    e = jnp.exp(s * 0.125 + bias)
    denom = jnp.sum(e, -1, keepdims=True)
    o3 = jax.lax.dot_general(
        e.astype(jnp.bfloat16), vg_ref[...], (((2,), (1,)), ((0,), (0,))),
        preferred_element_type=jnp.float32,
    )
    o3 = o3 * (1.0 / denom)
    for h in range(NH):
        o_ref[:, h * DH:(h + 1) * DH] = o3[h].astype(jnp.bfloat16)


def _attn(q, kt, v, mask_f, idx_flat, val_flat):
    grid_spec = pltpu.PrefetchScalarGridSpec(
        num_scalar_prefetch=2,
        grid=(NB,),
        in_specs=[
            pl.BlockSpec((NH, BS, DH), lambda n, *_: (0, n, 0)),
            pl.BlockSpec((NB, NH, DH, BS), lambda n, *_: (0, 0, 0, 0)),
            pl.BlockSpec((NH, S, DH), lambda n, *_: (0, 0, 0)),
            pl.BlockSpec((NB, 1, BS), lambda n, *_: (0, 0, 0)),
        ],
        out_specs=pl.BlockSpec((BS, H), lambda n, *_: (n, 0)),
        scratch_shapes=[
            pltpu.VMEM((NH, DH, NK * BS), jnp.bfloat16),
            pltpu.VMEM((NH, NK * BS, DH), jnp.bfloat16),
        ],
    )
    return pl.pallas_call(
        _attn_body,
        grid_spec=grid_spec,
        out_shape=jax.ShapeDtypeStruct((S, H), jnp.bfloat16),
    )(idx_flat, val_flat, q, kt, v, mask_f)


def _projln_body(o_ref, x_ref, w_ref, b_ref, g_ref, bb_ref, out_ref):
    a = jax.lax.dot_general(
        o_ref[...], w_ref[...], (((1,), (0,)), ((), ())),
        preferred_element_type=jnp.float32,
    )
    a = a + b_ref[...] + x_ref[...]
    out_ref[...] = _ln(a, g_ref[...], bb_ref[...])


def _projln(o, x, wo, bo, g, b):
    grid = (S // _QB,)
    return pl.pallas_call(
        _projln_body,
        grid=grid,
        in_specs=[
            pl.BlockSpec((_QB, H), lambda i: (i, 0)),
            pl.BlockSpec((_QB, H), lambda i: (i, 0)),
            pl.BlockSpec((H, H), lambda i: (0, 0)),
            pl.BlockSpec((1, H), lambda i: (0, 0)),
            pl.BlockSpec((1, H), lambda i: (0, 0)),
            pl.BlockSpec((1, H), lambda i: (0, 0)),
        ],
        out_specs=pl.BlockSpec((_QB, H), lambda i: (i, 0)),
        out_shape=jax.ShapeDtypeStruct((S, H), jnp.float32),
    )(o, x, wo, bo, g, b)


def _ffn_body(x_ref, w1_ref, b1_ref, w2_ref, b2_ref, g_ref, bb_ref, out_ref):
    xb = x_ref[...]
    h1 = jax.lax.dot_general(
        xb.astype(jnp.bfloat16), w1_ref[...], (((1,), (0,)), ((), ())),
        preferred_element_type=jnp.float32,
    )
    h1 = jax.nn.gelu(h1 + b1_ref[...])
    f = jax.lax.dot_general(
        h1.astype(jnp.bfloat16), w2_ref[...], (((1,), (0,)), ((), ())),
        preferred_element_type=jnp.float32,
    )
    f = f + b2_ref[...] + xb
    out_ref[...] = _ln(f, g_ref[...], bb_ref[...])


def _ffn(x, w1, b1, w2, b2, g, b):
    grid = (S // _QB,)
    return pl.pallas_call(
        _ffn_body,
        grid=grid,
        in_specs=[
            pl.BlockSpec((_QB, H), lambda i: (i, 0)),
            pl.BlockSpec((H, FF), lambda i: (0, 0)),
            pl.BlockSpec((1, FF), lambda i: (0, 0)),
            pl.BlockSpec((FF, H), lambda i: (0, 0)),
            pl.BlockSpec((1, H), lambda i: (0, 0)),
            pl.BlockSpec((1, H), lambda i: (0, 0)),
            pl.BlockSpec((1, H), lambda i: (0, 0)),
        ],
        out_specs=pl.BlockSpec((_QB, H), lambda i: (i, 0)),
        out_shape=jax.ShapeDtypeStruct((S, H), jnp.float32),
    )(x, w1, b1, w2, b2, g, b)


def kernel(word_ids, mask, type_ids, word_emb, pos_emb, type_emb, ln_emb_g,
           ln_emb_b, Wq, bq, Wk, bk, Wv, bv, Wo, bo, ln1_g, ln1_b, W1, b1,
           W2, b2, ln2_g, ln2_b):
    x = _gembed(
        word_ids.reshape(S),
        word_emb,
        pos_emb,
        type_ids.reshape(S, 1),
        type_emb,
        ln_emb_g.reshape(1, H),
        ln_emb_b.reshape(1, H),
    )
    mask_f = mask.reshape(NB, 1, BS).astype(jnp.float32)
    for l in range(L):
        idx, valid = _LAYOUTS[l]
        idx_flat = jnp.asarray(idx.reshape(-1), jnp.int32)
        val_flat = jnp.asarray(valid.reshape(-1).astype(np.int32))
        wqkv = jnp.concatenate(
            [Wq[l], Wk[l], Wv[l]], axis=1).astype(jnp.bfloat16)
        bqkv = jnp.concatenate([bq[l], bk[l], bv[l]]).reshape(1, 3 * H)
        q, kt, v = _qkv(x, wqkv, bqkv)
        o = _attn(q, kt, v, mask_f, idx_flat, val_flat)
        x = _projln(
            o, x, Wo[l].astype(jnp.bfloat16), bo[l].reshape(1, H),
            ln1_g[l].reshape(1, H), ln1_b[l].reshape(1, H))
        x = _ffn(
            x, W1[l].astype(jnp.bfloat16), b1[l].reshape(1, FF),
            W2[l].astype(jnp.bfloat16), b2[l].reshape(1, H),
            ln2_g[l].reshape(1, H), ln2_b[l].reshape(1, H))
    return x.reshape(B, S, H)


# speedup vs baseline: 1.0845x; 1.0845x over previous
import functools

import jax
import jax.numpy as jnp
import numpy as np
from jax.experimental import pallas as pl
from jax.experimental.pallas import tpu as pltpu

B, S, H, L, NH, DH = 1, 2048, 768, 2, 12, 64
V, TV, FF, BS, NR = 30522, 2, 3072, 64, 3
NB = S // BS
NK = 5 + NR


def _block_layout(nb, num_rand, seed):
    rng = np.random.RandomState(seed)
    idx = np.zeros((nb, 5 + num_rand), np.int32)
    valid = np.ones((nb, 5 + num_rand), np.float32)
    for i in range(nb):
        fixed = [0, nb - 1, (i - 1) % nb, i, (i + 1) % nb]
        rem = sorted(set(range(nb)) - set(fixed))
        r = rng.choice(rem, num_rand, replace=False)
        row = fixed + list(r)
        seen = set()
        for j, c in enumerate(row):
            idx[i, j] = c
            if c in seen:
                valid[i, j] = 0.0
            seen.add(c)
    return idx, valid


_LAYOUTS = [_block_layout(NB, NR, i) for i in range(L)]

def _ln(x, g, b):
    m = jnp.mean(x, -1, keepdims=True)
    v = jnp.mean((x - m) * (x - m), -1, keepdims=True)
    return (x - m) / jnp.sqrt(v + 1e-12) * g + b


def _gembed_body(ids_ref, tab_ref, pos_ref, tid_ref, te_ref, gg_ref, bb_ref,
                 o_ref, gath_ref, sem):
    def issue(t, _):
        for u in range(8):
            pltpu.make_async_copy(
                tab_ref.at[pl.ds(ids_ref[t * 8 + u], 1), :],
                gath_ref.at[pl.ds(t * 8 + u, 1), :],
                sem,
            ).start()
        return 0

    jax.lax.fori_loop(0, S // 8, issue, 0)

    def wait(t, _):
        pltpu.make_async_copy(
            tab_ref.at[pl.ds(0, 1), :], gath_ref.at[pl.ds(0, 1), :], sem
        ).wait()
        return 0

    jax.lax.fori_loop(0, S, wait, 0)

    x = gath_ref[...] + pos_ref[...]
    cond = tid_ref[...] == 0
    x = x + jnp.where(cond, te_ref[0:1, :], te_ref[1:2, :])
    o_ref[...] = _ln(x, gg_ref[...], bb_ref[...])


def _gembed(word_ids, word_emb, pos_emb, type_ids, type_emb, g, b):
    grid_spec = pltpu.PrefetchScalarGridSpec(
        num_scalar_prefetch=1,
        grid=(1,),
        in_specs=[
            pl.BlockSpec(memory_space=pl.ANY),
            pl.BlockSpec((S, H), lambda i, *_: (0, 0)),
            pl.BlockSpec((S, 1), lambda i, *_: (0, 0)),
            pl.BlockSpec((TV, H), lambda i, *_: (0, 0)),
            pl.BlockSpec((1, H), lambda i, *_: (0, 0)),
            pl.BlockSpec((1, H), lambda i, *_: (0, 0)),
        ],
        out_specs=pl.BlockSpec((S, H), lambda i, *_: (0, 0)),
        scratch_shapes=[
            pltpu.VMEM((S, H), jnp.float32),
            pltpu.SemaphoreType.DMA,
        ],
    )
    return pl.pallas_call(
        _gembed_body,
        grid_spec=grid_spec,
        out_shape=jax.ShapeDtypeStruct((S, H), jnp.float32),
    )(word_ids, word_emb, pos_emb, type_ids, type_emb, g, b)


_QB = 256
_NQ = S // _QB
_STEPS = _NQ + NB + _NQ


def _layer_body(idx_ref, val_ref, x_ref, wqkv_ref, bqkv_ref, mask_ref,
                wo_ref, bo_ref, g1_ref, b1_ref, w1_ref, bb1_ref,
                w2_ref, bb2_ref, g2_ref, b2_ref, out_ref,
                q_scr, kt_scr, v_scr, o_scr, kgt_scr, vg_scr):
    step = pl.program_id(0)

    @pl.when(step < _NQ)
    def _():
        i = step
        xb = x_ref[...].astype(jnp.bfloat16)
        r = jax.lax.dot_general(
            xb, wqkv_ref[...], (((1,), (0,)), ((), ())),
            preferred_element_type=jnp.float32)
        r = (r + bqkv_ref[...]).astype(jnp.bfloat16)
        for h in range(NH):
            q_scr[h, pl.ds(i * _QB, _QB), :] = r[:, h * DH:(h + 1) * DH]
            v_scr[h, pl.ds(i * _QB, _QB), :] = (
                r[:, 2 * H + h * DH:2 * H + (h + 1) * DH])
            for sb in range(_QB // BS):
                kt_scr[pl.ds(i * (_QB // BS) + sb, 1), h] = jnp.transpose(
                    r[sb * BS:(sb + 1) * BS,
                      H + h * DH:H + (h + 1) * DH])[None]

    @pl.when(jnp.logical_and(step >= _NQ, step < _NQ + NB))
    def _():
        n = step - _NQ
        bias_parts = []
        for j in range(NK):
            bi = idx_ref[n * NK + j]
            kgt_scr[:, :, j * BS:(j + 1) * BS] = kt_scr[bi]
            vg_scr[:, j * BS:(j + 1) * BS, :] = v_scr[:, pl.ds(bi * BS, BS), :]
            mv = mask_ref[bi]
            vj = val_ref[n * NK + j].astype(jnp.float32)
            bias_parts.append((1.0 - mv * vj) * (-1e9))
        bias = jnp.concatenate(bias_parts, axis=-1)[None]

        qb = q_scr[:, pl.ds(n * BS, BS), :]
        s = jax.lax.dot_general(
            qb, kgt_scr[...], (((2,), (1,)), ((0,), (0,))),
            preferred_element_type=jnp.float32)
        e = jnp.exp(s * 0.125 + bias)
        denom = jnp.sum(e, -1, keepdims=True)
        o3 = jax.lax.dot_general(
            e.astype(jnp.bfloat16), vg_scr[...], (((2,), (1,)), ((0,), (0,))),
            preferred_element_type=jnp.float32)
        o3 = o3 * (1.0 / denom)
        for h in range(NH):
            o_scr[pl.ds(n * BS, BS), h * DH:(h + 1) * DH] = (
                o3[h].astype(jnp.bfloat16))

    @pl.when(step >= _NQ + NB)
    def _():
        i = step - (_NQ + NB)
        ob = o_scr[pl.ds(i * _QB, _QB), :]
        a = jax.lax.dot_general(
            ob, wo_ref[...], (((1,), (0,)), ((), ())),
            preferred_element_type=jnp.float32)
        a = a + bo_ref[...] + x_ref[...]
        x1 = _ln(a, g1_ref[...], b1_ref[...])
        h1 = jax.lax.dot_general(
            x1.astype(jnp.bfloat16), w1_ref[...], (((1,), (0,)), ((), ())),
            preferred_element_type=jnp.float32)
        h1 = jax.nn.gelu(h1 + bb1_ref[...])
        f = jax.lax.dot_general(
            h1.astype(jnp.bfloat16), w2_ref[...], (((1,), (0,)), ((), ())),
            preferred_element_type=jnp.float32)
        f = f + bb2_ref[...] + x1
        out_ref[...] = _ln(f, g2_ref[...], b2_ref[...])


def _xmap(s, *_):
    return (jnp.where(s < _NQ, s,
                      jnp.where(s >= _NQ + NB, s - (_NQ + NB), 0)), 0)


def _layer(x, wqkv, bqkv, mask_f, idx_flat, val_flat,
           wo, bo, g1, b1, w1, bb1, w2, bb2, g2, b2):
    grid_spec = pltpu.PrefetchScalarGridSpec(
        num_scalar_prefetch=2,
        grid=(_STEPS,),
        in_specs=[
            pl.BlockSpec((_QB, H), _xmap),
            pl.BlockSpec((H, 3 * H), lambda s, *_: (0, 0)),
            pl.BlockSpec((1, 3 * H), lambda s, *_: (0, 0)),
            pl.BlockSpec((NB, 1, BS), lambda s, *_: (0, 0, 0)),
            pl.BlockSpec((H, H), lambda s, *_: (0, 0)),
            pl.BlockSpec((1, H), lambda s, *_: (0, 0)),
            pl.BlockSpec((1, H), lambda s, *_: (0, 0)),
            pl.BlockSpec((1, H), lambda s, *_: (0, 0)),
            pl.BlockSpec((H, FF), lambda s, *_: (0, 0)),
            pl.BlockSpec((1, FF), lambda s, *_: (0, 0)),
            pl.BlockSpec((FF, H), lambda s, *_: (0, 0)),
            pl.BlockSpec((1, H), lambda s, *_: (0, 0)),
            pl.BlockSpec((1, H), lambda s, *_: (0, 0)),
            pl.BlockSpec((1, H), lambda s, *_: (0, 0)),
        ],
        out_specs=pl.BlockSpec(
            (_QB, H),
            lambda s, *_: (jnp.where(s >= _NQ + NB, s - (_NQ + NB), 0), 0)),
        scratch_shapes=[
            pltpu.VMEM((NH, S, DH), jnp.bfloat16),
            pltpu.VMEM((NB, NH, DH, BS), jnp.bfloat16),
            pltpu.VMEM((NH, S, DH), jnp.bfloat16),
            pltpu.VMEM((S, H), jnp.bfloat16),
            pltpu.VMEM((NH, DH, NK * BS), jnp.bfloat16),
            pltpu.VMEM((NH, NK * BS, DH), jnp.bfloat16),
        ],
    )
    return pl.pallas_call(
        _layer_body,
        grid_spec=grid_spec,
        out_shape=jax.ShapeDtypeStruct((S, H), jnp.float32),
    )(idx_flat, val_flat, x, wqkv, bqkv, mask_f,
      wo, bo, g1, b1, w1, bb1, w2, bb2, g2, b2)


def kernel(word_ids, mask, type_ids, word_emb, pos_emb, type_emb, ln_emb_g,
           ln_emb_b, Wq, bq, Wk, bk, Wv, bv, Wo, bo, ln1_g, ln1_b, W1, b1,
           W2, b2, ln2_g, ln2_b):
    x = _gembed(
        word_ids.reshape(S),
        word_emb,
        pos_emb,
        type_ids.reshape(S, 1),
        type_emb,
        ln_emb_g.reshape(1, H),
        ln_emb_b.reshape(1, H),
    )
    mask_f = mask.reshape(NB, 1, BS).astype(jnp.float32)
    for l in range(L):
        idx, valid = _LAYOUTS[l]
        idx_flat = jnp.asarray(idx.reshape(-1), jnp.int32)
        val_flat = jnp.asarray(valid.reshape(-1).astype(np.int32))
        wqkv = jnp.concatenate(
            [Wq[l], Wk[l], Wv[l]], axis=1).astype(jnp.bfloat16)
        bqkv = jnp.concatenate([bq[l], bk[l], bv[l]]).reshape(1, 3 * H)
        x = _layer(
            x, wqkv, bqkv, mask_f, idx_flat, val_flat,
            Wo[l].astype(jnp.bfloat16), bo[l].reshape(1, H),
            ln1_g[l].reshape(1, H), ln1_b[l].reshape(1, H),
            W1[l].astype(jnp.bfloat16), b1[l].reshape(1, FF),
            W2[l].astype(jnp.bfloat16), b2[l].reshape(1, H),
            ln2_g[l].reshape(1, H), ln2_b[l].reshape(1, H))
    return x.reshape(B, S, H)


# speedup vs baseline: 1.1802x; 1.0883x over previous
import functools

import jax
import jax.numpy as jnp
import numpy as np
from jax.experimental import pallas as pl
from jax.experimental.pallas import tpu as pltpu

B, S, H, L, NH, DH = 1, 2048, 768, 2, 12, 64
V, TV, FF, BS, NR = 30522, 2, 3072, 64, 3
NB = S // BS
NK = 5 + NR


def _block_layout(nb, num_rand, seed):
    rng = np.random.RandomState(seed)
    idx = np.zeros((nb, 5 + num_rand), np.int32)
    valid = np.ones((nb, 5 + num_rand), np.float32)
    for i in range(nb):
        fixed = [0, nb - 1, (i - 1) % nb, i, (i + 1) % nb]
        rem = sorted(set(range(nb)) - set(fixed))
        r = rng.choice(rem, num_rand, replace=False)
        row = fixed + list(r)
        seen = set()
        for j, c in enumerate(row):
            idx[i, j] = c
            if c in seen:
                valid[i, j] = 0.0
            seen.add(c)
    return idx, valid


_LAYOUTS = [_block_layout(NB, NR, i) for i in range(L)]

def _ln(x, g, b):
    m = jnp.mean(x, -1, keepdims=True)
    v = jnp.mean((x - m) * (x - m), -1, keepdims=True)
    return (x - m) / jnp.sqrt(v + 1e-12) * g + b


def _gembed_body(ids_ref, tab_ref, pos_ref, tid_ref, te_ref, gg_ref, bb_ref,
                 o_ref, gath_ref, sem):
    def issue(t, _):
        for u in range(8):
            pltpu.make_async_copy(
                tab_ref.at[pl.ds(ids_ref[t * 8 + u], 1), :],
                gath_ref.at[pl.ds(t * 8 + u, 1), :],
                sem,
            ).start()
        return 0

    jax.lax.fori_loop(0, S // 8, issue, 0)

    def wait(t, _):
        for _u in range(16):
            pltpu.make_async_copy(
                tab_ref.at[pl.ds(0, 1), :], gath_ref.at[pl.ds(0, 1), :], sem
            ).wait()
        return 0

    jax.lax.fori_loop(0, S // 16, wait, 0)

    x = gath_ref[...] + pos_ref[...]
    cond = tid_ref[...] == 0
    x = x + jnp.where(cond, te_ref[0:1, :], te_ref[1:2, :])
    o_ref[...] = _ln(x, gg_ref[...], bb_ref[...])


def _gembed(word_ids, word_emb, pos_emb, type_ids, type_emb, g, b):
    grid_spec = pltpu.PrefetchScalarGridSpec(
        num_scalar_prefetch=1,
        grid=(1,),
        in_specs=[
            pl.BlockSpec(memory_space=pl.ANY),
            pl.BlockSpec((S, H), lambda i, *_: (0, 0)),
            pl.BlockSpec((S, 1), lambda i, *_: (0, 0)),
            pl.BlockSpec((TV, H), lambda i, *_: (0, 0)),
            pl.BlockSpec((1, H), lambda i, *_: (0, 0)),
            pl.BlockSpec((1, H), lambda i, *_: (0, 0)),
        ],
        out_specs=pl.BlockSpec((S, H), lambda i, *_: (0, 0)),
        scratch_shapes=[
            pltpu.VMEM((S, H), jnp.float32),
            pltpu.SemaphoreType.DMA,
        ],
    )
    return pl.pallas_call(
        _gembed_body,
        grid_spec=grid_spec,
        out_shape=jax.ShapeDtypeStruct((S, H), jnp.float32),
    )(word_ids, word_emb, pos_emb, type_ids, type_emb, g, b)


_QB = 256
_NQ = S // _QB
_AB = 2
_NA = NB // _AB
_STEPS = _NQ + _NA + _NQ


def _layer_body(idx_ref, val_ref, x_ref, wqkv_ref, bqkv_ref, mask_ref,
                wo_ref, bo_ref, g1_ref, b1_ref, w1_ref, bb1_ref,
                w2_ref, bb2_ref, g2_ref, b2_ref, out_ref,
                q_scr, kt_scr, v_scr, o_scr, kgt_scr, vg_scr):
    step = pl.program_id(0)

    @pl.when(step < _NQ)
    def _():
        i = step
        xb = x_ref[...].astype(jnp.bfloat16)
        r = jax.lax.dot_general(
            xb, wqkv_ref[...], (((1,), (0,)), ((), ())),
            preferred_element_type=jnp.float32)
        r = (r + bqkv_ref[...]).astype(jnp.bfloat16)
        for h in range(NH):
            q_scr[h, pl.ds(i * _QB, _QB), :] = r[:, h * DH:(h + 1) * DH]
            v_scr[h, pl.ds(i * _QB, _QB), :] = (
                r[:, 2 * H + h * DH:2 * H + (h + 1) * DH])
            for sb in range(_QB // BS):
                kt_scr[pl.ds(i * (_QB // BS) + sb, 1), h] = jnp.transpose(
                    r[sb * BS:(sb + 1) * BS,
                      H + h * DH:H + (h + 1) * DH])[None]

    @pl.when(jnp.logical_and(step >= _NQ, step < _NQ + _NA))
    def _():
        for local in range(_AB):
            n = (step - _NQ) * _AB + local
            bias_parts = []
            for j in range(NK):
                bi = idx_ref[n * NK + j]
                kgt_scr[local, :, :, j * BS:(j + 1) * BS] = kt_scr[bi]
                vg_scr[local, :, j * BS:(j + 1) * BS, :] = (
                    v_scr[:, pl.ds(bi * BS, BS), :])
                mv = mask_ref[bi]
                vj = val_ref[n * NK + j].astype(jnp.float32)
                bias_parts.append((1.0 - mv * vj) * (-1e9))
            bias = jnp.concatenate(bias_parts, axis=-1)[None]

            qb = q_scr[:, pl.ds(n * BS, BS), :]
            s = jax.lax.dot_general(
                qb, kgt_scr[local], (((2,), (1,)), ((0,), (0,))),
                preferred_element_type=jnp.float32)
            e = jnp.exp(s * 0.125 + bias)
            denom = jnp.sum(e, -1, keepdims=True)
            o3 = jax.lax.dot_general(
                e.astype(jnp.bfloat16), vg_scr[local],
                (((2,), (1,)), ((0,), (0,))),
                preferred_element_type=jnp.float32)
            o3 = o3 * (1.0 / denom)
            for h in range(NH):
                o_scr[pl.ds(n * BS, BS), h * DH:(h + 1) * DH] = (
                    o3[h].astype(jnp.bfloat16))

    @pl.when(step >= _NQ + _NA)
    def _():
        i = step - (_NQ + _NA)
        ob = o_scr[pl.ds(i * _QB, _QB), :]
        a = jax.lax.dot_general(
            ob, wo_ref[...], (((1,), (0,)), ((), ())),
            preferred_element_type=jnp.float32)
        a = a + bo_ref[...] + x_ref[...]
        x1 = _ln(a, g1_ref[...], b1_ref[...])
        h1 = jax.lax.dot_general(
            x1.astype(jnp.bfloat16), w1_ref[...], (((1,), (0,)), ((), ())),
            preferred_element_type=jnp.float32)
        h1 = jax.nn.gelu(h1 + bb1_ref[...])
        f = jax.lax.dot_general(
            h1.astype(jnp.bfloat16), w2_ref[...], (((1,), (0,)), ((), ())),
            preferred_element_type=jnp.float32)
        f = f + bb2_ref[...] + x1
        out_ref[...] = _ln(f, g2_ref[...], b2_ref[...])


def _xmap(s, *_):
    return (jnp.where(s < _NQ, s,
                      jnp.where(s >= _NQ + _NA, s - (_NQ + _NA), 0)), 0)


def _layer(x, wqkv, bqkv, mask_f, idx_flat, val_flat,
           wo, bo, g1, b1, w1, bb1, w2, bb2, g2, b2):
    grid_spec = pltpu.PrefetchScalarGridSpec(
        num_scalar_prefetch=2,
        grid=(_STEPS,),
        in_specs=[
            pl.BlockSpec((_QB, H), _xmap),
            pl.BlockSpec((H, 3 * H), lambda s, *_: (0, 0)),
            pl.BlockSpec((1, 3 * H), lambda s, *_: (0, 0)),
            pl.BlockSpec((NB, 1, BS), lambda s, *_: (0, 0, 0)),
            pl.BlockSpec((H, H), lambda s, *_: (0, 0)),
            pl.BlockSpec((1, H), lambda s, *_: (0, 0)),
            pl.BlockSpec((1, H), lambda s, *_: (0, 0)),
            pl.BlockSpec((1, H), lambda s, *_: (0, 0)),
            pl.BlockSpec((H, FF), lambda s, *_: (0, 0)),
            pl.BlockSpec((1, FF), lambda s, *_: (0, 0)),
            pl.BlockSpec((FF, H), lambda s, *_: (0, 0)),
            pl.BlockSpec((1, H), lambda s, *_: (0, 0)),
            pl.BlockSpec((1, H), lambda s, *_: (0, 0)),
            pl.BlockSpec((1, H), lambda s, *_: (0, 0)),
        ],
        out_specs=pl.BlockSpec(
            (_QB, H),
            lambda s, *_: (jnp.where(s >= _NQ + _NA, s - (_NQ + _NA), 0), 0)),
        scratch_shapes=[
            pltpu.VMEM((NH, S, DH), jnp.bfloat16),
            pltpu.VMEM((NB, NH, DH, BS), jnp.bfloat16),
            pltpu.VMEM((NH, S, DH), jnp.bfloat16),
            pltpu.VMEM((S, H), jnp.bfloat16),
            pltpu.VMEM((_AB, NH, DH, NK * BS), jnp.bfloat16),
            pltpu.VMEM((_AB, NH, NK * BS, DH), jnp.bfloat16),
        ],
    )
    return pl.pallas_call(
        _layer_body,
        grid_spec=grid_spec,
        out_shape=jax.ShapeDtypeStruct((S, H), jnp.float32),
    )(idx_flat, val_flat, x, wqkv, bqkv, mask_f,
      wo, bo, g1, b1, w1, bb1, w2, bb2, g2, b2)


def kernel(word_ids, mask, type_ids, word_emb, pos_emb, type_emb, ln_emb_g,
           ln_emb_b, Wq, bq, Wk, bk, Wv, bv, Wo, bo, ln1_g, ln1_b, W1, b1,
           W2, b2, ln2_g, ln2_b):
    x = _gembed(
        word_ids.reshape(S),
        word_emb,
        pos_emb,
        type_ids.reshape(S, 1),
        type_emb,
        ln_emb_g.reshape(1, H),
        ln_emb_b.reshape(1, H),
    )
    mask_f = mask.reshape(NB, 1, BS).astype(jnp.float32)
    for l in range(L):
        idx, valid = _LAYOUTS[l]
        idx_flat = jnp.asarray(idx.reshape(-1), jnp.int32)
        val_flat = jnp.asarray(valid.reshape(-1).astype(np.int32))
        wqkv = jnp.concatenate(
            [Wq[l], Wk[l], Wv[l]], axis=1).astype(jnp.bfloat16)
        bqkv = jnp.concatenate([bq[l], bk[l], bv[l]]).reshape(1, 3 * H)
        x = _layer(
            x, wqkv, bqkv, mask_f, idx_flat, val_flat,
            Wo[l].astype(jnp.bfloat16), bo[l].reshape(1, H),
            ln1_g[l].reshape(1, H), ln1_b[l].reshape(1, H),
            W1[l].astype(jnp.bfloat16), b1[l].reshape(1, FF),
            W2[l].astype(jnp.bfloat16), b2[l].reshape(1, H),
            ln2_g[l].reshape(1, H), ln2_b[l].reshape(1, H))
    return x.reshape(B, S, H)


# speedup vs baseline: 1.2472x; 1.0568x over previous
import functools

import jax
import jax.numpy as jnp
import numpy as np
from jax.experimental import pallas as pl
from jax.experimental.pallas import tpu as pltpu

B, S, H, L, NH, DH = 1, 2048, 768, 2, 12, 64
V, TV, FF, BS, NR = 30522, 2, 3072, 64, 3
NB = S // BS
NK = 5 + NR


def _block_layout(nb, num_rand, seed):
    rng = np.random.RandomState(seed)
    idx = np.zeros((nb, 5 + num_rand), np.int32)
    valid = np.ones((nb, 5 + num_rand), np.float32)
    for i in range(nb):
        fixed = [0, nb - 1, (i - 1) % nb, i, (i + 1) % nb]
        rem = sorted(set(range(nb)) - set(fixed))
        r = rng.choice(rem, num_rand, replace=False)
        row = fixed + list(r)
        seen = set()
        for j, c in enumerate(row):
            idx[i, j] = c
            if c in seen:
                valid[i, j] = 0.0
            seen.add(c)
    return idx, valid


_LAYOUTS = [_block_layout(NB, NR, i) for i in range(L)]

def _ln(x, g, b):
    m = jnp.mean(x, -1, keepdims=True)
    v = jnp.mean((x - m) * (x - m), -1, keepdims=True)
    return (x - m) / jnp.sqrt(v + 1e-12) * g + b


def _gembed_body(ids_ref, tab_ref, pos_ref, tid_ref, te_ref, gg_ref, bb_ref,
                 o_ref, gath_ref, sem):
    def issue(t, _):
        for u in range(8):
            pltpu.make_async_copy(
                tab_ref.at[pl.ds(ids_ref[t * 8 + u], 1), :],
                gath_ref.at[pl.ds(t * 8 + u, 1), :],
                sem,
            ).start()
        return 0

    jax.lax.fori_loop(0, S // 8, issue, 0)

    def wait(t, _):
        for _u in range(16):
            pltpu.make_async_copy(
                tab_ref.at[pl.ds(0, 1), :], gath_ref.at[pl.ds(0, 1), :], sem
            ).wait()
        return 0

    jax.lax.fori_loop(0, S // 16, wait, 0)

    x = gath_ref[...] + pos_ref[...]
    cond = tid_ref[...] == 0
    x = x + jnp.where(cond, te_ref[0:1, :], te_ref[1:2, :])
    o_ref[...] = _ln(x, gg_ref[...], bb_ref[...])


def _gembed(word_ids, word_emb, pos_emb, type_ids, type_emb, g, b):
    grid_spec = pltpu.PrefetchScalarGridSpec(
        num_scalar_prefetch=1,
        grid=(1,),
        in_specs=[
            pl.BlockSpec(memory_space=pl.ANY),
            pl.BlockSpec((S, H), lambda i, *_: (0, 0)),
            pl.BlockSpec((S, 1), lambda i, *_: (0, 0)),
            pl.BlockSpec((TV, H), lambda i, *_: (0, 0)),
            pl.BlockSpec((1, H), lambda i, *_: (0, 0)),
            pl.BlockSpec((1, H), lambda i, *_: (0, 0)),
        ],
        out_specs=pl.BlockSpec((S, H), lambda i, *_: (0, 0)),
        scratch_shapes=[
            pltpu.VMEM((S, H), jnp.float32),
            pltpu.SemaphoreType.DMA,
        ],
    )
    return pl.pallas_call(
        _gembed_body,
        grid_spec=grid_spec,
        out_shape=jax.ShapeDtypeStruct((S, H), jnp.float32),
    )(word_ids, word_emb, pos_emb, type_ids, type_emb, g, b)


_QB = 512
_NQ = S // _QB
_AB = 4
_NA = NB // _AB
_STEPS = _NQ + _NA + _NQ


def _layer_body(idx_ref, val_ref, x_ref, wqkv_ref, bqkv_ref, mask_ref,
                wo_ref, bo_ref, g1_ref, b1_ref, w1_ref, bb1_ref,
                w2_ref, bb2_ref, g2_ref, b2_ref, out_ref,
                q_scr, kt_scr, v_scr, o_scr, kgt_scr, vg_scr):
    step = pl.program_id(0)

    @pl.when(step < _NQ)
    def _():
        i = step
        xb = x_ref[...].astype(jnp.bfloat16)
        r = jax.lax.dot_general(
            xb, wqkv_ref[...], (((1,), (0,)), ((), ())),
            preferred_element_type=jnp.float32)
        r = (r + bqkv_ref[...]).astype(jnp.bfloat16)
        for h in range(NH):
            q_scr[h, pl.ds(i * _QB, _QB), :] = r[:, h * DH:(h + 1) * DH]
            v_scr[h, pl.ds(i * _QB, _QB), :] = (
                r[:, 2 * H + h * DH:2 * H + (h + 1) * DH])
            for sb in range(_QB // BS):
                kt_scr[pl.ds(i * (_QB // BS) + sb, 1), h] = jnp.transpose(
                    r[sb * BS:(sb + 1) * BS,
                      H + h * DH:H + (h + 1) * DH])[None]

    @pl.when(jnp.logical_and(step >= _NQ, step < _NQ + _NA))
    def _():
        for local in range(_AB):
            n = (step - _NQ) * _AB + local
            bias_parts = []
            for j in range(NK):
                bi = idx_ref[n * NK + j]
                kgt_scr[local, :, :, j * BS:(j + 1) * BS] = kt_scr[bi]
                vg_scr[local, :, j * BS:(j + 1) * BS, :] = (
                    v_scr[:, pl.ds(bi * BS, BS), :])
                mv = mask_ref[bi]
                vj = val_ref[n * NK + j].astype(jnp.float32)
                bias_parts.append((1.0 - mv * vj) * (-1e9))
            bias = jnp.concatenate(bias_parts, axis=-1)[None]

            qb = q_scr[:, pl.ds(n * BS, BS), :]
            s = jax.lax.dot_general(
                qb, kgt_scr[local], (((2,), (1,)), ((0,), (0,))),
                preferred_element_type=jnp.float32)
            e = jnp.exp(s * 0.125 + bias)
            denom = jnp.sum(e, -1, keepdims=True)
            o3 = jax.lax.dot_general(
                e.astype(jnp.bfloat16), vg_scr[local],
                (((2,), (1,)), ((0,), (0,))),
                preferred_element_type=jnp.float32)
            o3 = o3 * (1.0 / denom)
            for h in range(NH):
                o_scr[pl.ds(n * BS, BS), h * DH:(h + 1) * DH] = (
                    o3[h].astype(jnp.bfloat16))

    @pl.when(step >= _NQ + _NA)
    def _():
        i = step - (_NQ + _NA)
        ob = o_scr[pl.ds(i * _QB, _QB), :]
        a = jax.lax.dot_general(
            ob, wo_ref[...], (((1,), (0,)), ((), ())),
            preferred_element_type=jnp.float32)
        a = a + bo_ref[...] + x_ref[...]
        x1 = _ln(a, g1_ref[...], b1_ref[...])
        h1 = jax.lax.dot_general(
            x1.astype(jnp.bfloat16), w1_ref[...], (((1,), (0,)), ((), ())),
            preferred_element_type=jnp.float32)
        h1 = jax.nn.gelu(h1 + bb1_ref[...])
        f = jax.lax.dot_general(
            h1.astype(jnp.bfloat16), w2_ref[...], (((1,), (0,)), ((), ())),
            preferred_element_type=jnp.float32)
        f = f + bb2_ref[...] + x1
        out_ref[...] = _ln(f, g2_ref[...], b2_ref[...])


def _xmap(s, *_):
    return (jnp.where(s < _NQ, s,
                      jnp.where(s >= _NQ + _NA, s - (_NQ + _NA), 0)), 0)


def _layer(x, wqkv, bqkv, mask_f, idx_flat, val_flat,
           wo, bo, g1, b1, w1, bb1, w2, bb2, g2, b2):
    grid_spec = pltpu.PrefetchScalarGridSpec(
        num_scalar_prefetch=2,
        grid=(_STEPS,),
        in_specs=[
            pl.BlockSpec((_QB, H), _xmap),
            pl.BlockSpec((H, 3 * H), lambda s, *_: (0, 0)),
            pl.BlockSpec((1, 3 * H), lambda s, *_: (0, 0)),
            pl.BlockSpec((NB, 1, BS), lambda s, *_: (0, 0, 0)),
            pl.BlockSpec((H, H), lambda s, *_: (0, 0)),
            pl.BlockSpec((1, H), lambda s, *_: (0, 0)),
            pl.BlockSpec((1, H), lambda s, *_: (0, 0)),
            pl.BlockSpec((1, H), lambda s, *_: (0, 0)),
            pl.BlockSpec((H, FF), lambda s, *_: (0, 0)),
            pl.BlockSpec((1, FF), lambda s, *_: (0, 0)),
            pl.BlockSpec((FF, H), lambda s, *_: (0, 0)),
            pl.BlockSpec((1, H), lambda s, *_: (0, 0)),
            pl.BlockSpec((1, H), lambda s, *_: (0, 0)),
            pl.BlockSpec((1, H), lambda s, *_: (0, 0)),
        ],
        out_specs=pl.BlockSpec(
            (_QB, H),
            lambda s, *_: (jnp.where(s >= _NQ + _NA, s - (_NQ + _NA), 0), 0)),
        scratch_shapes=[
            pltpu.VMEM((NH, S, DH), jnp.bfloat16),
            pltpu.VMEM((NB, NH, DH, BS), jnp.bfloat16),
            pltpu.VMEM((NH, S, DH), jnp.bfloat16),
            pltpu.VMEM((S, H), jnp.bfloat16),
            pltpu.VMEM((_AB, NH, DH, NK * BS), jnp.bfloat16),
            pltpu.VMEM((_AB, NH, NK * BS, DH), jnp.bfloat16),
        ],
    )
    return pl.pallas_call(
        _layer_body,
        grid_spec=grid_spec,
        out_shape=jax.ShapeDtypeStruct((S, H), jnp.float32),
    )(idx_flat, val_flat, x, wqkv, bqkv, mask_f,
      wo, bo, g1, b1, w1, bb1, w2, bb2, g2, b2)


def kernel(word_ids, mask, type_ids, word_emb, pos_emb, type_emb, ln_emb_g,
           ln_emb_b, Wq, bq, Wk, bk, Wv, bv, Wo, bo, ln1_g, ln1_b, W1, b1,
           W2, b2, ln2_g, ln2_b):
    x = _gembed(
        word_ids.reshape(S),
        word_emb,
        pos_emb,
        type_ids.reshape(S, 1),
        type_emb,
        ln_emb_g.reshape(1, H),
        ln_emb_b.reshape(1, H),
    )
    mask_f = mask.reshape(NB, 1, BS).astype(jnp.float32)
    for l in range(L):
        idx, valid = _LAYOUTS[l]
        idx_flat = jnp.asarray(idx.reshape(-1), jnp.int32)
        val_flat = jnp.asarray(valid.reshape(-1).astype(np.int32))
        wqkv = jnp.concatenate(
            [Wq[l], Wk[l], Wv[l]], axis=1).astype(jnp.bfloat16)
        bqkv = jnp.concatenate([bq[l], bk[l], bv[l]]).reshape(1, 3 * H)
        x = _layer(
            x, wqkv, bqkv, mask_f, idx_flat, val_flat,
            Wo[l].astype(jnp.bfloat16), bo[l].reshape(1, H),
            ln1_g[l].reshape(1, H), ln1_b[l].reshape(1, H),
            W1[l].astype(jnp.bfloat16), b1[l].reshape(1, FF),
            W2[l].astype(jnp.bfloat16), b2[l].reshape(1, H),
            ln2_g[l].reshape(1, H), ln2_b[l].reshape(1, H))
    return x.reshape(B, S, H)
